# Initial kernel scaffold; baseline (speedup 1.0000x reference)
import jax
import jax.numpy as jnp
from jax.experimental import pallas as pl


def kernel(h, segment_ids, is_target, depth, feature, W, b):
    raise NotImplementedError("write your pallas kernel here")


# speedup vs baseline: 6.0346x; 6.0346x over previous
import functools

import jax
import jax.numpy as jnp
from jax import lax
from jax.experimental import pallas as pl
from jax.experimental.pallas import tpu as pltpu
from jax.experimental.pallas import tpu_sc as plsc

N = 32768
H = 128
B = 16
DAPP = 32
NCLS = 2

NC = 2
NS = 16
NW = NC * NS
RW = N // NW
CH = 128
NCH = RW // CH
L = 16


def _seg_sum_body(h_hbm, seg_hbm, tgt_hbm, out_hbm,
                  seg_v, tgt_v, idx_v, buf_v, zero_v, acc_sh, gsem):
    c = lax.axis_index("c")
    s = lax.axis_index("s")
    wid = s * NC + c
    base = wid * RW

    pltpu.sync_copy(seg_hbm.at[pl.ds(base, RW)], seg_v)
    pltpu.sync_copy(tgt_hbm.at[pl.ds(base, RW)], tgt_v)

    @pl.when(s == 0)
    def _zero():
        zv = jnp.zeros((L,), jnp.float32)

        def zrow(i, carry):
            zero_v[i // (H // L), pl.ds((i % (H // L)) * L, L)] = zv
            return carry

        lax.fori_loop(0, (B + 1) * (H // L), zrow, 0)
        pltpu.sync_copy(zero_v, acc_sh)

    trash = jnp.full((L,), B, jnp.int32)

    def mkidx(i, carry):
        sv = seg_v[pl.ds(i * L, L)]
        tv = tgt_v[pl.ds(i * L, L)]
        idx_v[i // (CH // L), pl.ds((i % (CH // L)) * L, L)] = jnp.where(
            tv == 1, sv, trash)
        return carry

    lax.fori_loop(0, RW // L, mkidx, 0)

    plsc.subcore_barrier()

    pltpu.async_copy(h_hbm.at[pl.ds(base, CH)], buf_v.at[0], gsem)
    for i in range(NCH):
        if i + 1 < NCH:
            pltpu.async_copy(h_hbm.at[pl.ds(base + (i + 1) * CH, CH)],
                             buf_v.at[(i + 1) % 2], gsem)
        pltpu.make_async_copy(h_hbm.at[pl.ds(base + i * CH, CH)],
                              buf_v.at[i % 2], gsem).wait()
        pltpu.sync_copy(buf_v.at[i % 2], acc_sh.at[idx_v.at[i]], add=True)

    plsc.subcore_barrier()

    @pl.when(s == 0)
    def _emit():
        pltpu.sync_copy(acc_sh, out_hbm.at[c])


@functools.lru_cache(maxsize=1)
def _seg_sum():
    return pl.kernel(
        _seg_sum_body,
        out_type=jax.ShapeDtypeStruct((NC, B + 1, H), jnp.float32),
        mesh=plsc.VectorSubcoreMesh(core_axis_name="c", subcore_axis_name="s"),
        scratch_types=[
            pltpu.VMEM((RW,), jnp.int32),
            pltpu.VMEM((RW,), jnp.int32),
            pltpu.VMEM((NCH, CH), jnp.int32),
            pltpu.VMEM((2, CH, H), jnp.float32),
            pltpu.VMEM((B + 1, H), jnp.float32),
            pltpu.VMEM_SHARED((B + 1, H), jnp.float32),
            pltpu.SemaphoreType.DMA,
        ],
    )


def _finalize_body(parts_ref, seg_ref, tgt_ref, dep_ref, feat_ref,
                   w1_ref, w2_ref, w3_ref, b_ref, out_ref):
    gh = parts_ref[0, :B, :] + parts_ref[1, :B, :]
    seg = seg_ref[...]
    tgt = tgt_ref[...]
    dep = dep_ref[...]
    gid = lax.broadcasted_iota(jnp.int32, (B,) + seg.shape, 0)
    m = seg[None, :, :] == gid
    num_tot = jnp.sum(m.astype(jnp.float32), axis=(1, 2))
    num_tgt = jnp.sum(jnp.where(jnp.logical_and(m, tgt[None, :, :] == 1),
                                1.0, 0.0), axis=(1, 2))
    mx = jnp.max(jnp.where(m, dep[None, :, :], -jnp.inf), axis=(1, 2))
    logits = (
        jnp.dot(gh, w1_ref[...], preferred_element_type=jnp.float32)
        + jnp.dot(feat_ref[...], w2_ref[...], preferred_element_type=jnp.float32)
        + mx[:, None] * w3_ref[0, :][None, :]
        + num_tgt[:, None] * w3_ref[1, :][None, :]
        + num_tot[:, None] * w3_ref[2, :][None, :]
        + b_ref[0, :][None, :]
    )
    out_ref[...] = logits


def kernel(h, segment_ids, is_target, depth, feature, W, b):
    seg = segment_ids.astype(jnp.int32)
    tgt = is_target.astype(jnp.int32)
    parts = _seg_sum()(h, seg, tgt)
    logits = pl.pallas_call(
        _finalize_body,
        out_shape=jax.ShapeDtypeStruct((B, NCLS), jnp.float32),
    )(parts, seg.reshape(N // H, H), tgt.reshape(N // H, H),
      depth.reshape(N // H, H), feature,
      W[:H], W[H:H + DAPP], W[H + DAPP:], b.reshape(1, NCLS))
    return logits
